# baseline (speedup 1.0000x reference)
import functools

import jax
import jax.numpy as jnp
import numpy as np
from jax import lax
from jax.experimental import pallas as pl
from jax.experimental.pallas import tpu as pltpu
from jax.experimental.pallas import tpu_sc as plsc

_L = 16


def _make_sc_kernel(bs, npts, G, C, nworkers):
    total_pts = bs * npts
    pts_per_tile = total_pts // nworkers
    CHUNK = 128
    GROUPS = CHUNK // _L
    nchunks = pts_per_tile // CHUNK
    G3 = G * G * G
    cs = np.float32(1.0) / np.float32(G - 1.0)
    eps = np.float32(1e-6)
    one_m_eps = np.float32(1.0) - eps

    mesh = plsc.VectorSubcoreMesh(core_axis_name="c", subcore_axis_name="s")

    @functools.partial(
        pl.kernel,
        out_type=(
            jax.ShapeDtypeStruct((total_pts * 8, C), jnp.float32),
            jax.ShapeDtypeStruct((total_pts * 8,), jnp.float32),
            jax.ShapeDtypeStruct((total_pts * 24,), jnp.float32),
        ),
        mesh=mesh,
        compiler_params=pltpu.CompilerParams(
            needs_layout_passes=False, use_tc_tiling_on_sc=False
        ),
        scratch_types=[
            pltpu.VMEM((pts_per_tile,), jnp.float32),
            pltpu.VMEM((pts_per_tile,), jnp.float32),
            pltpu.VMEM((pts_per_tile,), jnp.float32),
            pltpu.VMEM((GROUPS, 8 * _L), jnp.int32),
            pltpu.VMEM((CHUNK * 8, C), jnp.float32),
            pltpu.VMEM((CHUNK * 8,), jnp.float32),
            pltpu.VMEM((CHUNK * 24,), jnp.float32),
            pltpu.SemaphoreType.DMA,
        ],
    )
    def sc_kernel(table, ptst, lat_out, w_out, x_out,
                  sx, sy, sz, idxb, latbuf, wbuf, xbuf, sem):
        cid = lax.axis_index("c")
        sid = lax.axis_index("s")
        wid = sid * 2 + cid
        pt0 = wid * pts_per_tile
        b = pt0 // npts
        o = pt0 - b * npts
        gbase = b * G3

        pltpu.sync_copy(ptst.at[pl.ds((b * 3 + 0) * npts + o, pts_per_tile)], sx)
        pltpu.sync_copy(ptst.at[pl.ds((b * 3 + 1) * npts + o, pts_per_tile)], sy)
        pltpu.sync_copy(ptst.at[pl.ds((b * 3 + 2) * npts + o, pts_per_tile)], sz)

        iota = lax.iota(jnp.int32, _L)

        def chunk_body(k, carry):
            lp0 = k * CHUNK
            for g in range(GROUPS):
                base = lp0 + g * _L
                px = sx[pl.ds(base, _L)]
                py = sy[pl.ds(base, _L)]
                pz = sz[pl.ds(base, _L)]

                axes = []
                for p in (px, py, pz):
                    pc = jnp.minimum(jnp.maximum(p, eps), one_m_eps)
                    tf = pc / cs
                    ind = tf.astype(jnp.int32)
                    xyz0 = ind.astype(jnp.float32) * cs
                    x0 = (pc - xyz0) / cs
                    x1 = (pc - (xyz0 + cs)) / cs
                    axes.append((ind, x0, x1, jnp.abs(x1), jnp.abs(x0)))

                (ix, xx0, xx1, dx0, dx1) = axes[0]
                (iy, xy0, xy1, dy0, dy1) = axes[1]
                (iz, xz0, xz1, dz0, dz1) = axes[2]

                fb = gbase + (ix * G + iy) * G + iz
                pxy = (dx0 * dy0, dx0 * dy1, dx1 * dy0, dx1 * dy1)

                rowg = jnp.full((_L,), g, jnp.int32)
                col8 = iota * 8
                wpos = g * (8 * _L) + col8
                xpos = g * (24 * _L) + iota * 24
                for n in range(8):
                    nd, nh, nw = (n >> 2) & 1, (n >> 1) & 1, n & 1
                    cn = fb + (nd * G * G + nh * G + nw)
                    plsc.store_scatter(idxb, [rowg, col8 + n], cn)
                    wn = pxy[nd * 2 + nh] * (dz1 if nw else dz0)
                    plsc.store_scatter(wbuf, [wpos + n], wn)
                    plsc.store_scatter(xbuf, [xpos + 3 * n],
                                       xx1 if nd else xx0)
                    plsc.store_scatter(xbuf, [xpos + 3 * n + 1],
                                       xy1 if nh else xy0)
                    plsc.store_scatter(xbuf, [xpos + 3 * n + 2],
                                       xz1 if nw else xz0)

            cps = [
                pltpu.async_copy(
                    table.at[idxb.at[g]],
                    latbuf.at[pl.ds(g * 8 * _L, 8 * _L)],
                    sem,
                )
                for g in range(GROUPS)
            ]
            for cp in cps:
                cp.wait()

            ob = (pt0 + lp0) * 8
            pltpu.sync_copy(latbuf, lat_out.at[pl.ds(ob, CHUNK * 8)])
            pltpu.sync_copy(wbuf, w_out.at[pl.ds(ob, CHUNK * 8)])
            pltpu.sync_copy(xbuf, x_out.at[pl.ds(ob * 3, CHUNK * 24)])
            return carry

        lax.fori_loop(0, nchunks, chunk_body, 0)

    return sc_kernel


def kernel(grid, pts):
    bs, npts, _ = pts.shape
    G = grid.shape[1]
    C = grid.shape[-1]
    table = grid.reshape(bs * G * G * G, C)
    ptst = pts.transpose(0, 2, 1).reshape(bs * 3 * npts)
    sc = _make_sc_kernel(bs, npts, G, C, nworkers=32)
    lat_rows, w_flat, x_flat = sc(table, ptst)
    lat = lat_rows.reshape(bs, npts, 8, C)
    weight = w_flat.reshape(bs, npts, 8)
    xloc = x_flat.reshape(bs, npts, 8, 3)
    return lat, weight, xloc


# speedup vs baseline: 1.3444x; 1.3444x over previous
import functools

import jax
import jax.numpy as jnp
import numpy as np
from jax import lax
from jax.experimental import pallas as pl
from jax.experimental.pallas import tpu as pltpu
from jax.experimental.pallas import tpu_sc as plsc

_L = 16


def _make_sc_kernel(bs, npts, G, C, nworkers):
    total_pts = bs * npts
    pts_per_tile = total_pts // nworkers
    CHUNK = 128
    GROUPS = CHUNK // _L
    nchunks = pts_per_tile // CHUNK
    ptiles = npts // CHUNK
    G3 = G * G * G
    C8 = C // 8
    cs = np.float32(1.0) / np.float32(G - 1.0)
    eps = np.float32(1e-6)
    one_m_eps = np.float32(1.0) - eps

    mesh = plsc.VectorSubcoreMesh(core_axis_name="c", subcore_axis_name="s")

    @functools.partial(
        pl.kernel,
        out_type=(
            jax.ShapeDtypeStruct((bs, 8, C8, ptiles, 8, CHUNK), jnp.float32),
            jax.ShapeDtypeStruct((bs, ptiles, 8, CHUNK), jnp.float32),
            jax.ShapeDtypeStruct((bs, 3, ptiles, 8, CHUNK), jnp.float32),
        ),
        mesh=mesh,
        compiler_params=pltpu.CompilerParams(
            needs_layout_passes=False, use_tc_tiling_on_sc=False
        ),
        scratch_types=[
            pltpu.VMEM((pts_per_tile,), jnp.float32),
            pltpu.VMEM((pts_per_tile,), jnp.float32),
            pltpu.VMEM((pts_per_tile,), jnp.float32),
            pltpu.VMEM((8, CHUNK), jnp.int32),
            pltpu.VMEM((CHUNK * 8, C), jnp.float32),
            pltpu.VMEM((4 * C8, 8, CHUNK), jnp.float32),
            pltpu.VMEM((4 * C8, 8, CHUNK), jnp.float32),
            pltpu.VMEM((8, CHUNK), jnp.float32),
            pltpu.VMEM((3, 8, CHUNK), jnp.float32),
            pltpu.SemaphoreType.DMA,
            pltpu.SemaphoreType.DMA,
        ],
    )
    def sc_kernel(table, ptst, lat_out, w_out, x_out,
                  sx, sy, sz, idxb, latbuf, lt0, lt1, wbuf, xbuf, gsem, wsem):
        cid = lax.axis_index("c")
        sid = lax.axis_index("s")
        wid = sid * 2 + cid
        pt0 = wid * pts_per_tile
        b = pt0 // npts
        o = pt0 - b * npts
        tile0 = o // CHUNK
        gbase = b * G3

        pltpu.sync_copy(ptst.at[pl.ds((b * 3 + 0) * npts + o, pts_per_tile)], sx)
        pltpu.sync_copy(ptst.at[pl.ds((b * 3 + 1) * npts + o, pts_per_tile)], sy)
        pltpu.sync_copy(ptst.at[pl.ds((b * 3 + 2) * npts + o, pts_per_tile)], sz)

        iota = lax.iota(jnp.int32, _L)
        lts = (lt0, lt1)

        def chunk_body(k, carry):
            lp0 = k * CHUNK
            ptile = tile0 + k
            for g in range(GROUPS):
                base = lp0 + g * _L
                px = sx[pl.ds(base, _L)]
                py = sy[pl.ds(base, _L)]
                pz = sz[pl.ds(base, _L)]

                axes = []
                for p in (px, py, pz):
                    pc = jnp.minimum(jnp.maximum(p, eps), one_m_eps)
                    tf = pc / cs
                    ind = tf.astype(jnp.int32)
                    xyz0 = ind.astype(jnp.float32) * cs
                    x0 = (pc - xyz0) / cs
                    x1 = (pc - (xyz0 + cs)) / cs
                    axes.append((ind, x0, x1, jnp.abs(x1), jnp.abs(x0)))

                (ix, xx0, xx1, dx0, dx1) = axes[0]
                (iy, xy0, xy1, dy0, dy1) = axes[1]
                (iz, xz0, xz1, dz0, dz1) = axes[2]

                fb = gbase + (ix * G + iy) * G + iz
                pxy = (dx0 * dy0, dx0 * dy1, dx1 * dy0, dx1 * dy1)

                gsl = pl.ds(g * _L, _L)
                for n in range(8):
                    nd, nh, nw = (n >> 2) & 1, (n >> 1) & 1, n & 1
                    idxb[n, gsl] = fb + (nd * G * G + nh * G + nw)
                    wbuf[n, gsl] = pxy[nd * 2 + nh] * (dz1 if nw else dz0)
                    xbuf[0, n, gsl] = xx1 if nd else xx0
                    xbuf[1, n, gsl] = xy1 if nh else xy0
                    xbuf[2, n, gsl] = xz1 if nw else xz0

            gcps = [
                pltpu.async_copy(
                    table.at[idxb.at[n]],
                    latbuf.at[pl.ds(n * CHUNK, CHUNK)],
                    gsem,
                )
                for n in range(8)
            ]
            for cp in gcps:
                cp.wait()

            wcps = []
            for half in range(2):
                lt = lts[half]

                def tr_body(g, _, half=half, lt=lt):
                    g16 = g * _L + iota
                    for nn in range(4):
                        n = half * 4 + nn
                        rown = n * CHUNK + g16
                        for c in range(C):
                            val = plsc.load_gather(
                                latbuf, [rown, jnp.full((_L,), c, jnp.int32)]
                            )
                            lt[nn * C8 + (c // 8), c % 8, pl.ds(g * _L, _L)] = val
                    return _

                lax.fori_loop(0, GROUPS, tr_body, 0)
                for nn in range(4):
                    n = half * 4 + nn
                    for c8 in range(C8):
                        wcps.append(pltpu.async_copy(
                            lt.at[nn * C8 + c8],
                            lat_out.at[b, n, c8, ptile],
                            wsem,
                        ))

            wcps.append(pltpu.async_copy(wbuf, w_out.at[b, ptile], wsem))
            for a in range(3):
                wcps.append(pltpu.async_copy(
                    xbuf.at[a], x_out.at[b, a, ptile], wsem))
            for cp in wcps:
                cp.wait()
            return carry

        lax.fori_loop(0, nchunks, chunk_body, 0)

    return sc_kernel


def kernel(grid, pts):
    bs, npts, _ = pts.shape
    G = grid.shape[1]
    C = grid.shape[-1]
    table = grid.reshape(bs * G * G * G, C)
    ptst = pts.transpose(0, 2, 1).reshape(bs * 3 * npts)
    sc = _make_sc_kernel(bs, npts, G, C, nworkers=32)
    latP, wP, xP = sc(table, ptst)
    lat = latP.transpose(0, 3, 5, 1, 2, 4).reshape(bs, npts, 8, C)
    weight = wP.transpose(0, 1, 3, 2).reshape(bs, npts, 8)
    xloc = xP.transpose(0, 2, 4, 3, 1).reshape(bs, npts, 8, 3)
    return lat, weight, xloc


# speedup vs baseline: 3.8386x; 2.8553x over previous
import functools

import jax
import jax.numpy as jnp
import numpy as np
from jax import lax
from jax.experimental import pallas as pl
from jax.experimental.pallas import tpu as pltpu
from jax.experimental.pallas import tpu_sc as plsc

_L = 16
_TP = 129


def _make_sc_kernel(bs, npts, G, C, nworkers):
    total_pts = bs * npts
    pts_per_tile = total_pts // nworkers
    CHUNK = 128
    GROUPS = CHUNK // _L
    nchunks = pts_per_tile // CHUNK
    ptiles = npts // CHUNK
    G3 = G * G * G
    C8 = C // 8
    cs = np.float32(1.0) / np.float32(G - 1.0)
    eps = np.float32(1e-6)
    one_m_eps = np.float32(1.0) - eps

    mesh = plsc.VectorSubcoreMesh(core_axis_name="c", subcore_axis_name="s")

    @functools.partial(
        pl.kernel,
        out_type=(
            jax.ShapeDtypeStruct((bs, 8, C8, ptiles, 8, CHUNK), jnp.float32),
            jax.ShapeDtypeStruct((bs, ptiles, 8, CHUNK), jnp.float32),
            jax.ShapeDtypeStruct((bs, 3, ptiles, 8, CHUNK), jnp.float32),
        ),
        mesh=mesh,
        compiler_params=pltpu.CompilerParams(
            needs_layout_passes=False, use_tc_tiling_on_sc=False
        ),
        scratch_types=[
            pltpu.VMEM((pts_per_tile,), jnp.float32),
            pltpu.VMEM((pts_per_tile,), jnp.float32),
            pltpu.VMEM((pts_per_tile,), jnp.float32),
            pltpu.VMEM((8, CHUNK), jnp.int32),
            pltpu.VMEM((CHUNK * 8, C), jnp.float32),
            pltpu.VMEM((4 * C8, 8, _TP), jnp.float32),
            pltpu.VMEM((4 * C8, 8, _TP), jnp.float32),
            pltpu.VMEM((8, CHUNK), jnp.float32),
            pltpu.VMEM((3, 8, CHUNK), jnp.float32),
            pltpu.SemaphoreType.DMA,
            pltpu.SemaphoreType.DMA,
        ],
    )
    def sc_kernel(table, ptst, lat_out, w_out, x_out,
                  sx, sy, sz, idxb, latbuf, lt0, lt1, wbuf, xbuf, gsem, wsem):
        cid = lax.axis_index("c")
        sid = lax.axis_index("s")
        wid = sid * 2 + cid
        pt0 = wid * pts_per_tile
        b = pt0 // npts
        o = pt0 - b * npts
        tile0 = o // CHUNK
        gbase = b * G3

        pltpu.sync_copy(ptst.at[pl.ds((b * 3 + 0) * npts + o, pts_per_tile)], sx)
        pltpu.sync_copy(ptst.at[pl.ds((b * 3 + 1) * npts + o, pts_per_tile)], sy)
        pltpu.sync_copy(ptst.at[pl.ds((b * 3 + 2) * npts + o, pts_per_tile)], sz)

        iota = lax.iota(jnp.int32, _L)
        idiv8 = iota // 8
        imod8 = iota - idiv8 * 8
        lts = (lt0, lt1)

        def chunk_body(k, carry):
            lp0 = k * CHUNK
            ptile = tile0 + k
            for g in range(GROUPS):
                base = lp0 + g * _L
                px = sx[pl.ds(base, _L)]
                py = sy[pl.ds(base, _L)]
                pz = sz[pl.ds(base, _L)]

                axes = []
                for p in (px, py, pz):
                    pc = jnp.minimum(jnp.maximum(p, eps), one_m_eps)
                    tf = pc / cs
                    ind = tf.astype(jnp.int32)
                    xyz0 = ind.astype(jnp.float32) * cs
                    x0 = (pc - xyz0) / cs
                    x1 = (pc - (xyz0 + cs)) / cs
                    axes.append((ind, x0, x1, jnp.abs(x1), jnp.abs(x0)))

                (ix, xx0, xx1, dx0, dx1) = axes[0]
                (iy, xy0, xy1, dy0, dy1) = axes[1]
                (iz, xz0, xz1, dz0, dz1) = axes[2]

                fb = gbase + (ix * G + iy) * G + iz
                pxy = (dx0 * dy0, dx0 * dy1, dx1 * dy0, dx1 * dy1)

                gsl = pl.ds(g * _L, _L)
                for n in range(8):
                    nd, nh, nw = (n >> 2) & 1, (n >> 1) & 1, n & 1
                    idxb[n, gsl] = fb + (nd * G * G + nh * G + nw)
                    wbuf[n, gsl] = pxy[nd * 2 + nh] * (dz1 if nw else dz0)
                    xbuf[0, n, gsl] = xx1 if nd else xx0
                    xbuf[1, n, gsl] = xy1 if nh else xy0
                    xbuf[2, n, gsl] = xz1 if nw else xz0

            gcps = [
                pltpu.async_copy(
                    table.at[idxb.at[n]],
                    latbuf.at[pl.ds(n * CHUNK, CHUNK)],
                    gsem,
                )
                for n in range(8)
            ]
            wcps = [pltpu.async_copy(wbuf, w_out.at[b, ptile], wsem)]
            for a in range(3):
                wcps.append(pltpu.async_copy(
                    xbuf.at[a], x_out.at[b, a, ptile], wsem))
            for cp in gcps:
                cp.wait()

            @plsc.parallel_loop(0, CHUNK, 1, unroll=2)
            def tr_body(p):
                colp = jnp.full((_L,), 0, jnp.int32) + p
                for half in range(2):
                    lt = lts[half]
                    for nn in range(4):
                        n = half * 4 + nn
                        row = n * CHUNK + p
                        for h in range(2):
                            val = latbuf[row, pl.ds(h * _L, _L)]
                            plsc.store_scatter(
                                lt,
                                [idiv8 + (nn * C8 + 2 * h), imod8, colp],
                                val,
                            )

            for half in range(2):
                lt = lts[half]
                for nn in range(4):
                    n = half * 4 + nn
                    for c8 in range(C8):
                        wcps.append(pltpu.async_copy(
                            lt.at[nn * C8 + c8, :, pl.ds(0, CHUNK)],
                            lat_out.at[b, n, c8, ptile],
                            wsem,
                        ))
            for cp in wcps:
                cp.wait()
            return carry

        lax.fori_loop(0, nchunks, chunk_body, 0)

    return sc_kernel


def kernel(grid, pts):
    bs, npts, _ = pts.shape
    G = grid.shape[1]
    C = grid.shape[-1]
    table = grid.reshape(bs * G * G * G, C)
    ptst = pts.transpose(0, 2, 1).reshape(bs * 3 * npts)
    sc = _make_sc_kernel(bs, npts, G, C, nworkers=32)
    latP, wP, xP = sc(table, ptst)
    lat = latP.transpose(0, 3, 5, 1, 2, 4).reshape(bs, npts, 8, C)
    weight = wP.transpose(0, 1, 3, 2).reshape(bs, npts, 8)
    xloc = xP.transpose(0, 2, 4, 3, 1).reshape(bs, npts, 8, 3)
    return lat, weight, xloc


# speedup vs baseline: 5.6484x; 1.4715x over previous
import functools

import jax
import jax.numpy as jnp
import numpy as np
from jax import lax
from jax.experimental import pallas as pl
from jax.experimental.pallas import tpu as pltpu
from jax.experimental.pallas import tpu_sc as plsc

_L = 16
_TP = 129


def _make_sc_kernel(bs, npts, G, C, nworkers):
    total_pts = bs * npts
    pts_per_tile = total_pts // nworkers
    CHUNK = 128
    GROUPS = CHUNK // _L
    nchunks = pts_per_tile // CHUNK
    ptiles = npts // CHUNK
    G3 = G * G * G
    C8 = C // 8
    cs = np.float32(1.0) / np.float32(G - 1.0)
    eps = np.float32(1e-6)
    one_m_eps = np.float32(1.0) - eps

    mesh = plsc.VectorSubcoreMesh(core_axis_name="c", subcore_axis_name="s")

    @functools.partial(
        pl.kernel,
        out_type=(
            jax.ShapeDtypeStruct((bs, 8, C8, ptiles, 8, CHUNK), jnp.float32),
            jax.ShapeDtypeStruct((bs, ptiles, 8, CHUNK), jnp.float32),
            jax.ShapeDtypeStruct((bs, 3, ptiles, 8, CHUNK), jnp.float32),
        ),
        mesh=mesh,
        compiler_params=pltpu.CompilerParams(
            needs_layout_passes=False, use_tc_tiling_on_sc=False
        ),
        scratch_types=[
            pltpu.VMEM((pts_per_tile,), jnp.float32),
            pltpu.VMEM((pts_per_tile,), jnp.float32),
            pltpu.VMEM((pts_per_tile,), jnp.float32),
            pltpu.VMEM((8, CHUNK), jnp.int32),
            pltpu.VMEM((CHUNK * 8, C), jnp.float32),
            pltpu.VMEM((4 * C8, 8, _TP), jnp.float32),
            pltpu.VMEM((4 * C8, 8, _TP), jnp.float32),
            pltpu.VMEM((8, CHUNK), jnp.float32),
            pltpu.VMEM((3, 8, CHUNK), jnp.float32),
            pltpu.SemaphoreType.DMA,
            pltpu.SemaphoreType.DMA,
        ],
    )
    def sc_kernel(table, ptst, lat_out, w_out, x_out,
                  sx, sy, sz, idxb, latbuf, lt0, lt1, wbuf, xbuf, gsem, wsem):
        cid = lax.axis_index("c")
        sid = lax.axis_index("s")
        wid = sid * 2 + cid
        pt0 = wid * pts_per_tile
        b = pt0 // npts
        o = pt0 - b * npts
        tile0 = o // CHUNK
        gbase = b * G3

        pltpu.sync_copy(ptst.at[pl.ds((b * 3 + 0) * npts + o, pts_per_tile)], sx)
        pltpu.sync_copy(ptst.at[pl.ds((b * 3 + 1) * npts + o, pts_per_tile)], sy)
        pltpu.sync_copy(ptst.at[pl.ds((b * 3 + 2) * npts + o, pts_per_tile)], sz)

        iota = lax.iota(jnp.int32, _L)
        idiv8 = iota // 8
        imod8 = iota - idiv8 * 8
        lts = (lt0, lt1)

        def chunk_body(k, carry):
            lp0 = k * CHUNK
            ptile = tile0 + k
            for g in range(GROUPS):
                base = lp0 + g * _L
                px = sx[pl.ds(base, _L)]
                py = sy[pl.ds(base, _L)]
                pz = sz[pl.ds(base, _L)]

                axes = []
                for p in (px, py, pz):
                    pc = jnp.minimum(jnp.maximum(p, eps), one_m_eps)
                    tf = pc / cs
                    ind = tf.astype(jnp.int32)
                    xyz0 = ind.astype(jnp.float32) * cs
                    x0 = (pc - xyz0) / cs
                    x1 = (pc - (xyz0 + cs)) / cs
                    axes.append((ind, x0, x1, jnp.abs(x1), jnp.abs(x0)))

                (ix, xx0, xx1, dx0, dx1) = axes[0]
                (iy, xy0, xy1, dy0, dy1) = axes[1]
                (iz, xz0, xz1, dz0, dz1) = axes[2]

                dp = (ix * (G * G), ix * (G * G) + G * G)
                iy1 = iy + 1
                iz1 = iz + 1
                WLT = 128 // C
                hp = ((iy // 8) * (8 * G) + (iy % 8) * WLT,
                      (iy1 // 8) * (8 * G) + (iy1 % 8) * WLT)
                wp = ((iz // WLT) * (8 * WLT) + iz % WLT,
                      (iz1 // WLT) * (8 * WLT) + iz1 % WLT)
                pxy = (dx0 * dy0, dx0 * dy1, dx1 * dy0, dx1 * dy1)

                gsl = pl.ds(g * _L, _L)
                for n in range(8):
                    nd, nh, nw = (n >> 2) & 1, (n >> 1) & 1, n & 1
                    idxb[n, gsl] = gbase + dp[nd] + hp[nh] + wp[nw]
                    wbuf[n, gsl] = pxy[nd * 2 + nh] * (dz1 if nw else dz0)
                    xbuf[0, n, gsl] = xx1 if nd else xx0
                    xbuf[1, n, gsl] = xy1 if nh else xy0
                    xbuf[2, n, gsl] = xz1 if nw else xz0

            gcps = [
                pltpu.async_copy(
                    table.at[idxb.at[n]],
                    latbuf.at[pl.ds(n * CHUNK, CHUNK)],
                    gsem,
                )
                for n in range(8)
            ]
            wcps = [pltpu.async_copy(wbuf, w_out.at[b, ptile], wsem)]
            for a in range(3):
                wcps.append(pltpu.async_copy(
                    xbuf.at[a], x_out.at[b, a, ptile], wsem))
            for cp in gcps:
                cp.wait()

            @plsc.parallel_loop(0, CHUNK, 1, unroll=2)
            def tr_body(p):
                colp = jnp.full((_L,), 0, jnp.int32) + p
                for half in range(2):
                    lt = lts[half]
                    for nn in range(4):
                        n = half * 4 + nn
                        row = n * CHUNK + p
                        for h in range(2):
                            val = latbuf[row, pl.ds(h * _L, _L)]
                            plsc.store_scatter(
                                lt,
                                [idiv8 + (nn * C8 + 2 * h), imod8, colp],
                                val,
                            )

            for half in range(2):
                lt = lts[half]
                for nn in range(4):
                    n = half * 4 + nn
                    for c8 in range(C8):
                        wcps.append(pltpu.async_copy(
                            lt.at[nn * C8 + c8, :, pl.ds(0, CHUNK)],
                            lat_out.at[b, n, c8, ptile],
                            wsem,
                        ))
            for cp in wcps:
                cp.wait()
            return carry

        lax.fori_loop(0, nchunks, chunk_body, 0)

    return sc_kernel


def kernel(grid, pts):
    bs, npts, _ = pts.shape
    G = grid.shape[1]
    C = grid.shape[-1]
    g6 = grid.reshape(bs, G, G // 8, 8, G * C // 128, 128)
    g6 = g6.transpose(0, 1, 2, 4, 3, 5)
    table = g6.reshape(bs * G * G * G, C)
    ptst = pts.transpose(0, 2, 1).reshape(bs * 3 * npts)
    sc = _make_sc_kernel(bs, npts, G, C, nworkers=32)
    latP, wP, xP = sc(table, ptst)
    lat = latP.transpose(0, 3, 5, 1, 2, 4).reshape(bs, npts, 8, C)
    weight = wP.transpose(0, 1, 3, 2).reshape(bs, npts, 8)
    xloc = xP.transpose(0, 2, 4, 3, 1).reshape(bs, npts, 8, 3)
    return lat, weight, xloc
